# baseline (speedup 1.0000x reference)
import functools

import jax
import jax.numpy as jnp
from jax import lax
from jax.experimental import pallas as pl
from jax.experimental.pallas import tpu as pltpu
from jax.experimental.pallas import tpu_sc as plsc

D_MODEL = 64
NEG_K = 5
K_TOT = NEG_K + 1

NC = 2
NS = 16
LANES = 16
NW = NC * NS


def _sigmoid(x):
    return 1.0 / (1.0 + jnp.exp(-x))


@functools.partial(jax.jit, static_argnames=("batch", "vocab"))
def _sc_scores(h_flat, sup_idx, half_off, table2, batch, vocab):
    bw = batch // NW
    ngrp = bw // LANES
    d2 = 2 * D_MODEL

    mesh = plsc.VectorSubcoreMesh(core_axis_name="c", subcore_axis_name="s")

    @functools.partial(
        pl.kernel,
        mesh=mesh,
        compiler_params=pltpu.CompilerParams(needs_layout_passes=False),
        out_type=[
            jax.ShapeDtypeStruct((batch,), jnp.float32),
            jax.ShapeDtypeStruct((NEG_K * batch,), jnp.float32),
        ],
        scratch_types=[
            pltpu.VMEM((K_TOT, bw), jnp.int32),
            pltpu.VMEM((K_TOT, bw), jnp.int32),
            pltpu.VMEM((K_TOT * bw, d2), jnp.float32),
            pltpu.VMEM((bw * D_MODEL,), jnp.float32),
            pltpu.VMEM((K_TOT, bw), jnp.float32),
            pltpu.SemaphoreType.DMA,
        ],
    )
    def sc_fn(h_hbm, sup_hbm, half_hbm, table_hbm,
              pos_hbm, negout_hbm, idx_v, half_v, rows_v, h_v, out_v, sem):
        wid = lax.axis_index("s") * NC + lax.axis_index("c")
        base = wid * bw

        for k in range(K_TOT):
            pltpu.sync_copy(sup_hbm.at[pl.ds(k * batch + base, bw)],
                            idx_v.at[k])

        copies = [
            pltpu.async_copy(table_hbm.at[idx_v.at[k]],
                             rows_v.at[pl.ds(k * bw, bw)], sem)
            for k in range(K_TOT)
        ]
        pltpu.sync_copy(h_hbm.at[pl.ds(base * D_MODEL, bw * D_MODEL)], h_v)
        for k in range(K_TOT):
            pltpu.sync_copy(half_hbm.at[pl.ds(k * batch + base, bw)],
                            half_v.at[k])
        for cp in copies:
            cp.wait()

        iot = lax.iota(jnp.int32, LANES)
        for g in range(ngrp):
            l0 = g * LANES
            hbase = (l0 + iot) * D_MODEL
            rowis = [iot + (k * bw + l0) for k in range(K_TOT)]
            halfs = [half_v[k, pl.ds(l0, LANES)] for k in range(K_TOT)]

            def dbody(d, accs, hbase=hbase, rowis=rowis, halfs=halfs):
                hv = plsc.load_gather(h_v, [hbase + d])
                return tuple(
                    accs[k] + hv * plsc.load_gather(
                        rows_v, [rowis[k], halfs[k] + d])
                    for k in range(K_TOT)
                )

            accs = lax.fori_loop(
                0, D_MODEL, dbody,
                tuple(jnp.zeros((LANES,), jnp.float32) for _ in range(K_TOT)))
            for k in range(K_TOT):
                out_v[k, pl.ds(l0, LANES)] = _sigmoid(accs[k])

        pltpu.sync_copy(out_v.at[0], pos_hbm.at[pl.ds(base, bw)])
        for k in range(NEG_K):
            pltpu.sync_copy(out_v.at[k + 1],
                            negout_hbm.at[pl.ds(k * batch + base, bw)])

    return sc_fn(h_flat, sup_idx, half_off, table2)


def kernel(h, target_index, neg_index, emb_table):
    batch = h.shape[0]
    vocab = emb_table.shape[0]
    all_idx = jnp.concatenate(
        [target_index.astype(jnp.int32).reshape(1, batch),
         neg_index.astype(jnp.int32).T], axis=0).reshape(-1)
    sup_idx = all_idx >> 1
    half_off = (all_idx & 1) * D_MODEL
    table2 = emb_table.reshape(vocab // 2, 2 * D_MODEL)
    pos, neg_to = _sc_scores(h.reshape(-1), sup_idx, half_off, table2,
                             batch, vocab)
    pos_out = pos.reshape(batch, 1)
    neg_out = neg_to.reshape(NEG_K, batch).T
    pos_label = jnp.ones((batch, 1), dtype=jnp.float32)
    neg_label = jnp.zeros((batch, NEG_K), dtype=jnp.float32)
    return (pos_out, pos_label, neg_out, neg_label)


# speedup vs baseline: 1.3634x; 1.3634x over previous
import functools

import jax
import jax.numpy as jnp
from jax import lax
from jax.experimental import pallas as pl
from jax.experimental.pallas import tpu as pltpu
from jax.experimental.pallas import tpu_sc as plsc

D_MODEL = 64
NEG_K = 5
K_TOT = NEG_K + 1

NC = 2
NS = 16
LANES = 16
NW = NC * NS


def _sigmoid(x):
    return 1.0 / (1.0 + jnp.exp(-x))


@functools.partial(jax.jit, static_argnames=("batch",))
def _sc_scores(h_t, widx, emb_table, batch):
    bw = batch // NW
    ngrp = bw // LANES

    mesh = plsc.VectorSubcoreMesh(core_axis_name="c", subcore_axis_name="s")

    @functools.partial(
        pl.kernel,
        mesh=mesh,
        compiler_params=pltpu.CompilerParams(
            needs_layout_passes=False, use_tc_tiling_on_sc=False),
        out_type=[
            jax.ShapeDtypeStruct((batch,), jnp.float32),
            jax.ShapeDtypeStruct((NEG_K * batch,), jnp.float32),
        ],
        scratch_types=[
            pltpu.VMEM((K_TOT, bw), jnp.int32),
            pltpu.VMEM((K_TOT * bw, D_MODEL), jnp.float32),
            pltpu.VMEM((D_MODEL, bw), jnp.float32),
            pltpu.VMEM((K_TOT, bw), jnp.float32),
            pltpu.SemaphoreType.DMA,
        ],
    )
    def sc_fn(h_t_hbm, widx_hbm, table_hbm,
              pos_hbm, negout_hbm, idx_v, rows_v, h_v, out_v, sem):
        wid = lax.axis_index("s") * NC + lax.axis_index("c")
        base = wid * bw

        pltpu.sync_copy(widx_hbm.at[wid], idx_v)

        copies = [
            pltpu.async_copy(table_hbm.at[idx_v.at[k]],
                             rows_v.at[pl.ds(k * bw, bw)], sem)
            for k in range(K_TOT)
        ]
        pltpu.sync_copy(h_t_hbm.at[:, pl.ds(base, bw)], h_v)
        for cp in copies:
            cp.wait()

        iot = lax.iota(jnp.int32, LANES)
        for g in range(ngrp):
            l0 = g * LANES
            lanev = iot + l0
            rowis = [iot + (k * bw + l0) for k in range(K_TOT)]

            def dbody(d, accs, lanev=lanev, rowis=rowis):
                m = (iot + d) & (D_MODEL - 1)
                hv = plsc.load_gather(h_v, [m, lanev])
                return tuple(
                    accs[k] + hv * plsc.load_gather(rows_v, [rowis[k], m])
                    for k in range(K_TOT)
                )

            accs = lax.fori_loop(
                0, D_MODEL, dbody,
                tuple(jnp.zeros((LANES,), jnp.float32) for _ in range(K_TOT)))
            for k in range(K_TOT):
                out_v[k, pl.ds(l0, LANES)] = _sigmoid(accs[k])

        pltpu.sync_copy(out_v.at[0], pos_hbm.at[pl.ds(base, bw)])
        for k in range(NEG_K):
            pltpu.sync_copy(out_v.at[k + 1],
                            negout_hbm.at[pl.ds(k * batch + base, bw)])

    return sc_fn(h_t, widx, emb_table)


def kernel(h, target_index, neg_index, emb_table):
    batch = h.shape[0]
    bw = batch // NW
    tgt_w = target_index.astype(jnp.int32).reshape(NW, 1, bw)
    neg_w = (neg_index.astype(jnp.int32).T.reshape(NEG_K, NW, bw)
             .transpose(1, 0, 2))
    widx = jnp.concatenate([tgt_w, neg_w], axis=1)
    h_t = h.T
    pos, neg_to = _sc_scores(h_t, widx, emb_table, batch)
    pos_out = pos.reshape(batch, 1)
    neg_out = neg_to.reshape(NEG_K, batch).T
    pos_label = jnp.ones((batch, 1), dtype=jnp.float32)
    neg_label = jnp.zeros((batch, NEG_K), dtype=jnp.float32)
    return (pos_out, pos_label, neg_out, neg_label)
